# Initial kernel scaffold; baseline (speedup 1.0000x reference)
import jax
import jax.numpy as jnp
from jax.experimental import pallas as pl


def kernel(list_neigh, Imagetype_map, atom_type, ImageDR, num_neigh, nghost, c_param, W0, b0, W1, b1, W2, b2, W3, b3):
    raise NotImplementedError("write your pallas kernel here")


# speedup vs baseline: 2.6730x; 2.6730x over previous
import functools
import jax
import jax.numpy as jnp
from jax.experimental import pallas as pl
from jax.experimental.pallas import tpu as pltpu

B, N, T, M = 1, 20000, 2, 64
BETA, M1, M2 = 8, 8, 4
RMIN, RMAX = 0.5, 6.0
NFEAT = M1 * M2
TM = T * M

BN_A = 256
BN_B = 512
NPAD = 20480


def _stage_a_kernel(dr_ref, nn_ref, typ_ref, c0_ref, c1_ref,
                    feat_ref, stats_ref):
    pid = pl.program_id(0)

    @pl.when(pid == 0)
    def _init():
        stats_ref[...] = jnp.zeros_like(stats_ref)

    dr = dr_ref[...]
    bn = dr.shape[1]
    r = dr[0:TM, :]
    dx = dr[TM:2 * TM, :]
    dy = dr[2 * TM:3 * TM, :]
    dz = dr[3 * TM:4 * TM, :]

    row = jax.lax.broadcasted_iota(jnp.int32, (TM, bn), 0)
    nn = nn_ref[...]
    nn_sel = jnp.where(row < M, nn[0:1, :], nn[1:2, :])
    mask = ((row % M) < nn_sel).astype(jnp.float32)

    u = (r - RMIN) * (1.0 / (RMAX - RMIN))
    x = jnp.clip(2.0 * u - 1.0, -1.0, 1.0)
    fc = jnp.where(r < RMIN, 1.0,
                   jnp.where(r > RMAX, 0.0,
                             0.5 * jnp.cos(jnp.pi * jnp.clip(u, 0.0, 1.0)) + 0.5))
    fcm = fc * mask

    basis = [fcm, x * fcm]
    tprev, tcur = jnp.ones_like(x), x
    for _ in range(2, BETA):
        tprev, tcur = tcur, 2.0 * x * tcur - tprev
        basis.append(tcur * fcm)

    rinv = 1.0 / jnp.where(r > 1e-6, r, 1.0)
    sx = dx * rinv * mask
    sy = dy * rinv * mask
    sz = dz * rinv * mask
    svec = [mask, sx, sy, sz]

    g_rows = []
    for c in range(4):
        s = svec[c]
        for t in range(T):
            lo, hi = t * M, (t + 1) * M
            for k in range(BETA):
                p = basis[k][lo:hi, :] if c == 0 else basis[k][lo:hi, :] * s[lo:hi, :]
                g_rows.append(jnp.sum(p, axis=0, keepdims=True))
    G = jnp.concatenate(g_rows, axis=0)

    D0 = jnp.dot(c0_ref[...], G, preferred_element_type=jnp.float32)
    D1 = jnp.dot(c1_ref[...], G, preferred_element_type=jnp.float32)
    typ = typ_ref[...]
    D = jnp.where(typ == 0, D0, D1)

    f_parts = []
    for p in range(M2):
        acc = None
        for c in range(4):
            blk = D[c * M1:(c + 1) * M1, :] * D[c * M1 + p:c * M1 + p + 1, :]
            acc = blk if acc is None else acc + blk
        f_parts.append(acc)
    feat = jnp.concatenate(f_parts, axis=0)
    feat_ref[...] = feat

    colsum = jnp.sum(feat, axis=0, keepdims=True)
    colsq = jnp.sum(feat * feat, axis=0, keepdims=True)
    m0 = (typ == 0).astype(jnp.float32)
    m1 = (typ == 1).astype(jnp.float32)

    def fold(v):
        out = v[:, 0:128]
        for i in range(1, bn // 128):
            out = out + v[:, i * 128:(i + 1) * 128]
        return out

    upd = jnp.concatenate([
        fold(colsum * m0), fold(colsum * m1),
        fold(colsq * m0), fold(colsq * m1),
        fold(m0), fold(m1),
        jnp.zeros((2, 128), jnp.float32)], axis=0)
    stats_ref[...] += upd


def _stage_b_kernel(feat_ref, typ_ref, stats_ref,
                    w0_ref, b0_ref, w1_ref, b1_ref, w2_ref, b2_ref,
                    w3_ref, b3_ref, ei_ref, etot_ref):
    pid = pl.program_id(0)

    @pl.when(pid == 0)
    def _init():
        etot_ref[...] = jnp.zeros_like(etot_ref)

    stats = stats_ref[...]
    s0 = jnp.sum(stats[0, :])
    s1 = jnp.sum(stats[1, :])
    q0 = jnp.sum(stats[2, :])
    q1 = jnp.sum(stats[3, :])
    c0 = jnp.sum(stats[4, :]) * float(NFEAT)
    c1 = jnp.sum(stats[5, :]) * float(NFEAT)
    mean0 = s0 / jnp.maximum(c0, 1.0)
    mean1 = s1 / jnp.maximum(c1, 1.0)
    var0 = (q0 - c0 * mean0 * mean0) / jnp.maximum(c0 - 1.0, 1.0)
    var1 = (q1 - c1 * mean1 * mean1) / jnp.maximum(c1 - 1.0, 1.0)
    std0 = jnp.sqrt(jnp.maximum(var0, 0.0))
    std1 = jnp.sqrt(jnp.maximum(var1, 0.0))

    typ = typ_ref[...]
    is0 = (typ == 0)
    mean_a = jnp.where(is0, mean0, mean1)
    inv_a = jnp.where(is0, 1.0 / (std0 + 1e-12), 1.0 / (std1 + 1e-12))
    featn = (feat_ref[...] - mean_a) * inv_a

    def mlp(t):
        h = jnp.tanh(jnp.dot(w0_ref[t], featn,
                             preferred_element_type=jnp.float32) + b0_ref[t])
        h = jnp.tanh(jnp.dot(w1_ref[t], h,
                             preferred_element_type=jnp.float32) + b1_ref[t])
        h = jnp.tanh(jnp.dot(w2_ref[t], h,
                             preferred_element_type=jnp.float32) + b2_ref[t])
        return jnp.dot(w3_ref[t], h,
                       preferred_element_type=jnp.float32) + b3_ref[t]

    e0 = mlp(0)
    e1 = mlp(1)
    ei = jnp.where(is0, e0[0:1, :], jnp.where(typ == 1, e1[0:1, :], 0.0))
    ei_ref[...] = ei
    etot_ref[...] = etot_ref[...] + jnp.sum(ei, axis=1, keepdims=True)


@jax.jit
def _run(ImageDR, Imagetype_map, num_neigh, c_param,
         W0, b0, W1, b1, W2, b2, W3, b3):
    f32 = jnp.float32
    drT = jnp.transpose(ImageDR.reshape(N, TM, 4).astype(f32), (2, 1, 0))
    drT = drT.reshape(4 * TM, N)
    drT = jnp.pad(drT, ((0, 0), (0, NPAD - N)))
    nnT = jnp.pad(num_neigh.reshape(N, T).astype(jnp.int32).T,
                  ((0, 0), (0, NPAD - N)))
    typT = jnp.pad(Imagetype_map.astype(jnp.int32)[None, :],
                   ((0, 0), (0, NPAD - N)), constant_values=2)

    cp = c_param.astype(f32)
    blk = jnp.transpose(cp, (0, 2, 1, 3)).reshape(T, M1, T * BETA)
    eye4 = jnp.eye(4, dtype=f32)
    Cmat = jnp.einsum('ab,tmk->tambk', eye4, blk).reshape(T, 4 * M1, 4 * T * BETA)
    C0, C1 = Cmat[0], Cmat[1]

    W0p = W0.astype(f32).reshape(T, M1, M2, 50).transpose(0, 2, 1, 3).reshape(T, NFEAT, 50)
    W0T = jnp.transpose(W0p, (0, 2, 1))
    W1T = jnp.transpose(W1.astype(f32), (0, 2, 1))
    W2T = jnp.transpose(W2.astype(f32), (0, 2, 1))
    W3T = jnp.pad(jnp.transpose(W3.astype(f32), (0, 2, 1)), ((0, 0), (0, 7), (0, 0)))
    b0c = b0.astype(f32)[:, :, None]
    b1c = b1.astype(f32)[:, :, None]
    b2c = b2.astype(f32)[:, :, None]
    b3c = jnp.pad(b3.astype(f32)[:, :, None], ((0, 0), (0, 7), (0, 0)))

    grid_a = NPAD // BN_A
    featT, stats = pl.pallas_call(
        _stage_a_kernel,
        grid=(grid_a,),
        in_specs=[
            pl.BlockSpec((4 * TM, BN_A), lambda i: (0, i)),
            pl.BlockSpec((T, BN_A), lambda i: (0, i)),
            pl.BlockSpec((1, BN_A), lambda i: (0, i)),
            pl.BlockSpec((4 * M1, 4 * T * BETA), lambda i: (0, 0)),
            pl.BlockSpec((4 * M1, 4 * T * BETA), lambda i: (0, 0)),
        ],
        out_specs=[
            pl.BlockSpec((NFEAT, BN_A), lambda i: (0, i)),
            pl.BlockSpec((8, 128), lambda i: (0, 0)),
        ],
        out_shape=[
            jax.ShapeDtypeStruct((NFEAT, NPAD), f32),
            jax.ShapeDtypeStruct((8, 128), f32),
        ],
    )(drT, nnT, typT, C0, C1)

    grid_b = NPAD // BN_B
    ei, etot = pl.pallas_call(
        _stage_b_kernel,
        grid=(grid_b,),
        in_specs=[
            pl.BlockSpec((NFEAT, BN_B), lambda i: (0, i)),
            pl.BlockSpec((1, BN_B), lambda i: (0, i)),
            pl.BlockSpec((8, 128), lambda i: (0, 0)),
            pl.BlockSpec((T, 50, NFEAT), lambda i: (0, 0, 0)),
            pl.BlockSpec((T, 50, 1), lambda i: (0, 0, 0)),
            pl.BlockSpec((T, 50, 50), lambda i: (0, 0, 0)),
            pl.BlockSpec((T, 50, 1), lambda i: (0, 0, 0)),
            pl.BlockSpec((T, 50, 50), lambda i: (0, 0, 0)),
            pl.BlockSpec((T, 50, 1), lambda i: (0, 0, 0)),
            pl.BlockSpec((T, 8, 50), lambda i: (0, 0, 0)),
            pl.BlockSpec((T, 8, 1), lambda i: (0, 0, 0)),
        ],
        out_specs=[
            pl.BlockSpec((1, BN_B), lambda i: (0, i)),
            pl.BlockSpec((1, 1), lambda i: (0, 0)),
        ],
        out_shape=[
            jax.ShapeDtypeStruct((1, NPAD), f32),
            jax.ShapeDtypeStruct((1, 1), f32),
        ],
    )(featT, typT, stats, W0T, b0c, W1T, b1c, W2T, b2c, W3T, b3c)

    Ei = ei[:, :N].reshape(B, N)
    Etot = etot.reshape(B)
    return Etot, Ei


def kernel(list_neigh, Imagetype_map, atom_type, ImageDR, num_neigh, nghost,
           c_param, W0, b0, W1, b1, W2, b2, W3, b3):
    return _run(ImageDR, Imagetype_map, num_neigh, c_param,
                W0, b0, W1, b1, W2, b2, W3, b3)


# speedup vs baseline: 3.6854x; 1.3788x over previous
import functools
import jax
import jax.numpy as jnp
from jax.experimental import pallas as pl
from jax.experimental.pallas import tpu as pltpu

B, N, T, M = 1, 20000, 2, 64
BETA, M1, M2 = 8, 8, 4
RMIN, RMAX = 0.5, 6.0
NFEAT = M1 * M2
TM = T * M

BN_A = 256
BN_B = 2048
NPAD = 20480


def _stage_a_kernel(dr_ref, nn_ref, typ_ref, c0_ref, c1_ref,
                    feat_ref, stats_ref):
    pid = pl.program_id(0)

    @pl.when(pid == 0)
    def _init():
        stats_ref[...] = jnp.zeros_like(stats_ref)

    dr = dr_ref[...]
    bn = dr.shape[1]
    r = dr[0:TM, :]
    dx = dr[TM:2 * TM, :]
    dy = dr[2 * TM:3 * TM, :]
    dz = dr[3 * TM:4 * TM, :]

    row = jax.lax.broadcasted_iota(jnp.int32, (TM, bn), 0)
    nn = nn_ref[...]
    nn_sel = jnp.where(row < M, nn[0:1, :], nn[1:2, :])
    mask = ((row % M) < nn_sel).astype(jnp.float32)

    u = (r - RMIN) * (1.0 / (RMAX - RMIN))
    x = jnp.clip(2.0 * u - 1.0, -1.0, 1.0)
    w = jnp.clip(u, 0.0, 1.0) - 0.5
    pw = jnp.pi * w
    z = pw * pw
    sinpw = pw * (1.0 + z * (-1.0 / 6.0 + z * (1.0 / 120.0 + z * (
        -1.0 / 5040.0 + z * (1.0 / 362880.0 - z / 39916800.0)))))
    fcm = (0.5 - 0.5 * sinpw) * mask

    basis = [fcm, x * fcm]
    tprev, tcur = jnp.ones_like(x), x
    for _ in range(2, BETA):
        tprev, tcur = tcur, 2.0 * x * tcur - tprev
        basis.append(tcur * fcm)

    rinv = 1.0 / jnp.where(r > 1e-6, r, 1.0)
    svec = [None, dx * rinv, dy * rinv, dz * rinv]

    def fold8(p):
        p = p[0:32, :] + p[32:64, :]
        p = p[0:16, :] + p[16:32, :]
        return p[0:8, :] + p[8:16, :]

    g_parts = []
    for c in range(4):
        s = svec[c]
        for t in range(T):
            lo, hi = t * M, (t + 1) * M
            for k in range(BETA):
                p = basis[k][lo:hi, :] if c == 0 else basis[k][lo:hi, :] * s[lo:hi, :]
                g_parts.append(fold8(p))
    G = jnp.concatenate(g_parts, axis=0)

    D0 = jnp.dot(c0_ref[...], G, preferred_element_type=jnp.float32)
    D1 = jnp.dot(c1_ref[...], G, preferred_element_type=jnp.float32)
    typ = typ_ref[...]
    D = jnp.where(typ == 0, D0, D1)

    f_parts = []
    for p in range(M2):
        acc = None
        for c in range(4):
            blk = D[c * M1:(c + 1) * M1, :] * D[c * M1 + p:c * M1 + p + 1, :]
            acc = blk if acc is None else acc + blk
        f_parts.append(acc)
    feat = jnp.concatenate(f_parts, axis=0)
    feat_ref[...] = feat

    colsum = jnp.sum(feat, axis=0, keepdims=True)
    colsq = jnp.sum(feat * feat, axis=0, keepdims=True)
    m0 = (typ == 0).astype(jnp.float32)
    m1 = (typ == 1).astype(jnp.float32)

    def fold(v):
        out = v[:, 0:128]
        for i in range(1, bn // 128):
            out = out + v[:, i * 128:(i + 1) * 128]
        return out

    upd = jnp.concatenate([
        fold(colsum * m0), fold(colsum * m1),
        fold(colsq * m0), fold(colsq * m1),
        fold(m0), fold(m1),
        jnp.zeros((2, 128), jnp.float32)], axis=0)
    stats_ref[...] += upd


def _stage_b_kernel(feat_ref, typ_ref, stats_ref,
                    w0_ref, b0_ref, w1_ref, b1_ref, w2_ref, b2_ref,
                    w3_ref, b3_ref, ei_ref, etot_ref):
    pid = pl.program_id(0)

    @pl.when(pid == 0)
    def _init():
        etot_ref[...] = jnp.zeros_like(etot_ref)

    stats = stats_ref[...]
    s0 = jnp.sum(stats[0, :])
    s1 = jnp.sum(stats[1, :])
    q0 = jnp.sum(stats[2, :])
    q1 = jnp.sum(stats[3, :])
    c0 = jnp.sum(stats[4, :]) * float(NFEAT)
    c1 = jnp.sum(stats[5, :]) * float(NFEAT)
    mean0 = s0 / jnp.maximum(c0, 1.0)
    mean1 = s1 / jnp.maximum(c1, 1.0)
    var0 = (q0 - c0 * mean0 * mean0) / jnp.maximum(c0 - 1.0, 1.0)
    var1 = (q1 - c1 * mean1 * mean1) / jnp.maximum(c1 - 1.0, 1.0)
    std0 = jnp.sqrt(jnp.maximum(var0, 0.0))
    std1 = jnp.sqrt(jnp.maximum(var1, 0.0))

    typ = typ_ref[...]
    is0 = (typ == 0)
    mean_a = jnp.where(is0, mean0, mean1)
    inv_a = jnp.where(is0, 1.0 / (std0 + 1e-12), 1.0 / (std1 + 1e-12))
    featn = (feat_ref[...] - mean_a) * inv_a

    def mlp(t):
        h = jnp.tanh(jnp.dot(w0_ref[t], featn,
                             preferred_element_type=jnp.float32) + b0_ref[t])
        h = jnp.tanh(jnp.dot(w1_ref[t], h,
                             preferred_element_type=jnp.float32) + b1_ref[t])
        h = jnp.tanh(jnp.dot(w2_ref[t], h,
                             preferred_element_type=jnp.float32) + b2_ref[t])
        return jnp.dot(w3_ref[t], h,
                       preferred_element_type=jnp.float32) + b3_ref[t]

    e0 = mlp(0)
    e1 = mlp(1)
    ei = jnp.where(is0, e0[0:1, :], jnp.where(typ == 1, e1[0:1, :], 0.0))
    ei_ref[...] = ei
    etot_ref[...] = etot_ref[...] + jnp.sum(ei, axis=1, keepdims=True)


@jax.jit
def _run(ImageDR, Imagetype_map, num_neigh, c_param,
         W0, b0, W1, b1, W2, b2, W3, b3):
    f32 = jnp.float32
    drT = jnp.transpose(ImageDR.reshape(N, TM, 4).astype(f32), (2, 1, 0))
    drT = drT.reshape(4 * TM, N)
    drT = jnp.pad(drT, ((0, 0), (0, NPAD - N)))
    nnT = jnp.pad(num_neigh.reshape(N, T).astype(jnp.int32).T,
                  ((0, 0), (0, NPAD - N)))
    typT = jnp.pad(Imagetype_map.astype(jnp.int32)[None, :],
                   ((0, 0), (0, NPAD - N)), constant_values=2)

    cp = c_param.astype(f32)
    blk = jnp.transpose(cp, (0, 2, 1, 3)).reshape(T, M1, T * BETA)
    rep = jnp.repeat(blk, 8, axis=2)
    eye4 = jnp.eye(4, dtype=f32)
    Cmat = (eye4[None, :, None, :, None] * rep[:, None, :, None, :])
    Cmat = Cmat.reshape(T, 4 * M1, 4 * T * BETA * 8)
    C0, C1 = Cmat[0], Cmat[1]

    W0p = W0.astype(f32).reshape(T, M1, M2, 50).transpose(0, 2, 1, 3).reshape(T, NFEAT, 50)
    W0T = jnp.transpose(W0p, (0, 2, 1))
    W1T = jnp.transpose(W1.astype(f32), (0, 2, 1))
    W2T = jnp.transpose(W2.astype(f32), (0, 2, 1))
    W3T = jnp.pad(jnp.transpose(W3.astype(f32), (0, 2, 1)), ((0, 0), (0, 7), (0, 0)))
    b0c = b0.astype(f32)[:, :, None]
    b1c = b1.astype(f32)[:, :, None]
    b2c = b2.astype(f32)[:, :, None]
    b3c = jnp.pad(b3.astype(f32)[:, :, None], ((0, 0), (0, 7), (0, 0)))

    grid_a = NPAD // BN_A
    featT, stats = pl.pallas_call(
        _stage_a_kernel,
        grid=(grid_a,),
        in_specs=[
            pl.BlockSpec((4 * TM, BN_A), lambda i: (0, i)),
            pl.BlockSpec((T, BN_A), lambda i: (0, i)),
            pl.BlockSpec((1, BN_A), lambda i: (0, i)),
            pl.BlockSpec((4 * M1, 4 * T * BETA * 8), lambda i: (0, 0)),
            pl.BlockSpec((4 * M1, 4 * T * BETA * 8), lambda i: (0, 0)),
        ],
        out_specs=[
            pl.BlockSpec((NFEAT, BN_A), lambda i: (0, i)),
            pl.BlockSpec((8, 128), lambda i: (0, 0)),
        ],
        out_shape=[
            jax.ShapeDtypeStruct((NFEAT, NPAD), f32),
            jax.ShapeDtypeStruct((8, 128), f32),
        ],
    )(drT, nnT, typT, C0, C1)

    grid_b = NPAD // BN_B
    ei, etot = pl.pallas_call(
        _stage_b_kernel,
        grid=(grid_b,),
        in_specs=[
            pl.BlockSpec((NFEAT, BN_B), lambda i: (0, i)),
            pl.BlockSpec((1, BN_B), lambda i: (0, i)),
            pl.BlockSpec((8, 128), lambda i: (0, 0)),
            pl.BlockSpec((T, 50, NFEAT), lambda i: (0, 0, 0)),
            pl.BlockSpec((T, 50, 1), lambda i: (0, 0, 0)),
            pl.BlockSpec((T, 50, 50), lambda i: (0, 0, 0)),
            pl.BlockSpec((T, 50, 1), lambda i: (0, 0, 0)),
            pl.BlockSpec((T, 50, 50), lambda i: (0, 0, 0)),
            pl.BlockSpec((T, 50, 1), lambda i: (0, 0, 0)),
            pl.BlockSpec((T, 8, 50), lambda i: (0, 0, 0)),
            pl.BlockSpec((T, 8, 1), lambda i: (0, 0, 0)),
        ],
        out_specs=[
            pl.BlockSpec((1, BN_B), lambda i: (0, i)),
            pl.BlockSpec((1, 1), lambda i: (0, 0)),
        ],
        out_shape=[
            jax.ShapeDtypeStruct((1, NPAD), f32),
            jax.ShapeDtypeStruct((1, 1), f32),
        ],
    )(featT, typT, stats, W0T, b0c, W1T, b1c, W2T, b2c, W3T, b3c)

    Ei = ei[:, :N].reshape(B, N)
    Etot = etot.reshape(B)
    return Etot, Ei


def kernel(list_neigh, Imagetype_map, atom_type, ImageDR, num_neigh, nghost,
           c_param, W0, b0, W1, b1, W2, b2, W3, b3):
    return _run(ImageDR, Imagetype_map, num_neigh, c_param,
                W0, b0, W1, b1, W2, b2, W3, b3)


# speedup vs baseline: 4.1342x; 1.1218x over previous
import functools
import jax
import jax.numpy as jnp
from jax.experimental import pallas as pl
from jax.experimental.pallas import tpu as pltpu

B, N, T, M = 1, 20000, 2, 64
BETA, M1, M2 = 8, 8, 4
RMIN, RMAX = 0.5, 6.0
NFEAT = M1 * M2
TM = T * M

BN_A = 512
BN_B = 2560


def _stage_a_kernel(slab_n, dr_ref, nn_ref, typ_ref, c0_ref, c1_ref,
                    feat_ref, stats_ref):
    pid = pl.program_id(0)

    @pl.when(pid == 0)
    def _init():
        stats_ref[...] = jnp.zeros_like(stats_ref)

    dr = dr_ref[...]
    bn = dr.shape[1]
    valid = (pid * BN_A + jax.lax.broadcasted_iota(jnp.int32, (1, bn), 1)) < slab_n
    r = dr[0:TM, :]
    dx = dr[TM:2 * TM, :]
    dy = dr[2 * TM:3 * TM, :]
    dz = dr[3 * TM:4 * TM, :]

    row = jax.lax.broadcasted_iota(jnp.int32, (TM, bn), 0)
    nn = nn_ref[...]
    nn_sel = jnp.where(row < M, nn[0:1, :], nn[1:2, :])
    mask = ((row % M) < nn_sel).astype(jnp.float32)

    u = (r - RMIN) * (1.0 / (RMAX - RMIN))
    x = jnp.clip(2.0 * u - 1.0, -1.0, 1.0)
    w = jnp.clip(u, 0.0, 1.0) - 0.5
    pw = jnp.pi * w
    z = pw * pw
    sinpw = pw * (1.0 + z * (-1.0 / 6.0 + z * (1.0 / 120.0 + z * (
        -1.0 / 5040.0 + z * (1.0 / 362880.0 - z / 39916800.0)))))
    fcm = (0.5 - 0.5 * sinpw) * mask

    basis = [fcm, x * fcm]
    tprev, tcur = jnp.ones_like(x), x
    for _ in range(2, BETA):
        tprev, tcur = tcur, 2.0 * x * tcur - tprev
        basis.append(tcur * fcm)

    rinv = 1.0 / jnp.where(r > 1e-6, r, 1.0)
    svec = [None, dx * rinv, dy * rinv, dz * rinv]

    def fold8(p):
        p = p[0:32, :] + p[32:64, :]
        p = p[0:16, :] + p[16:32, :]
        return p[0:8, :] + p[8:16, :]

    g_parts = []
    for c in range(4):
        s = svec[c]
        for t in range(T):
            lo, hi = t * M, (t + 1) * M
            for k in range(BETA):
                p = basis[k][lo:hi, :] if c == 0 else basis[k][lo:hi, :] * s[lo:hi, :]
                g_parts.append(fold8(p))
    G = jnp.concatenate(g_parts, axis=0)

    D0 = jnp.dot(c0_ref[...], G, preferred_element_type=jnp.float32)
    D1 = jnp.dot(c1_ref[...], G, preferred_element_type=jnp.float32)
    typ = typ_ref[...]
    D = jnp.where(typ == 0, D0, D1)

    f_parts = []
    for p in range(M2):
        acc = None
        for c in range(4):
            blk = D[c * M1:(c + 1) * M1, :] * D[c * M1 + p:c * M1 + p + 1, :]
            acc = blk if acc is None else acc + blk
        f_parts.append(acc)
    feat = jnp.concatenate(f_parts, axis=0)
    feat = jnp.where(valid, feat, 0.0)
    feat_ref[...] = feat

    colsum = jnp.sum(feat, axis=0, keepdims=True)
    colsq = jnp.sum(feat * feat, axis=0, keepdims=True)
    m0 = jnp.where(valid & (typ == 0), 1.0, 0.0)
    m1 = jnp.where(valid & (typ == 1), 1.0, 0.0)
    upd = jnp.concatenate([
        colsum * m0, colsum * m1,
        colsq * m0, colsq * m1,
        m0, m1,
        jnp.zeros((2, bn), jnp.float32)], axis=0)
    stats_ref[...] += upd


def _stage_b_kernel(feat_ref, typ_ref, stats_ref,
                    w0_ref, b0_ref, w1_ref, b1_ref, w2_ref, b2_ref,
                    w3_ref, b3_ref, ei_ref, etot_ref):
    pid = pl.program_id(0)

    @pl.when(pid == 0)
    def _init():
        etot_ref[...] = jnp.zeros_like(etot_ref)

    stats = stats_ref[...]
    nrow = stats.shape[0]
    def _row(i):
        return sum(jnp.sum(stats[j, :]) for j in range(i, nrow, 8))
    s0 = _row(0)
    s1 = _row(1)
    q0 = _row(2)
    q1 = _row(3)
    c0 = _row(4) * float(NFEAT)
    c1 = _row(5) * float(NFEAT)
    mean0 = s0 / jnp.maximum(c0, 1.0)
    mean1 = s1 / jnp.maximum(c1, 1.0)
    var0 = (q0 - c0 * mean0 * mean0) / jnp.maximum(c0 - 1.0, 1.0)
    var1 = (q1 - c1 * mean1 * mean1) / jnp.maximum(c1 - 1.0, 1.0)
    std0 = jnp.sqrt(jnp.maximum(var0, 0.0))
    std1 = jnp.sqrt(jnp.maximum(var1, 0.0))

    typ = typ_ref[...]
    is0 = (typ == 0)
    mean_a = jnp.where(is0, mean0, mean1)
    inv_a = jnp.where(is0, 1.0 / (std0 + 1e-12), 1.0 / (std1 + 1e-12))
    featn = (feat_ref[...] - mean_a) * inv_a

    def mlp(t):
        h = jnp.tanh(jnp.dot(w0_ref[t], featn,
                             preferred_element_type=jnp.float32) + b0_ref[t])
        h = jnp.tanh(jnp.dot(w1_ref[t], h,
                             preferred_element_type=jnp.float32) + b1_ref[t])
        h = jnp.tanh(jnp.dot(w2_ref[t], h,
                             preferred_element_type=jnp.float32) + b2_ref[t])
        return jnp.dot(w3_ref[t], h,
                       preferred_element_type=jnp.float32) + b3_ref[t]

    e0 = mlp(0)
    e1 = mlp(1)
    bn = typ.shape[1]
    valid = (pid * BN_B + jax.lax.broadcasted_iota(jnp.int32, (1, bn), 1)) < N
    ei = jnp.where(is0, e0[0:1, :], jnp.where(typ == 1, e1[0:1, :], 0.0))
    ei = jnp.where(valid, ei, 0.0)
    ei_ref[...] = ei
    etot_ref[...] = etot_ref[...] + jnp.sum(ei, axis=1, keepdims=True)


@jax.jit
def _run(ImageDR, Imagetype_map, num_neigh, c_param,
         W0, b0, W1, b1, W2, b2, W3, b3):
    f32 = jnp.float32
    typT = Imagetype_map.astype(jnp.int32)[None, :]

    cp = c_param.astype(f32)
    blk = jnp.transpose(cp, (0, 2, 1, 3)).reshape(T, M1, T * BETA)
    rep = jnp.repeat(blk, 8, axis=2)
    eye4 = jnp.eye(4, dtype=f32)
    Cmat = (eye4[None, :, None, :, None] * rep[:, None, :, None, :])
    Cmat = Cmat.reshape(T, 4 * M1, 4 * T * BETA * 8)
    C0, C1 = Cmat[0], Cmat[1]

    W0p = W0.astype(f32).reshape(T, M1, M2, 50).transpose(0, 2, 1, 3).reshape(T, NFEAT, 50)
    W0T = jnp.transpose(W0p, (0, 2, 1))
    W1T = jnp.transpose(W1.astype(f32), (0, 2, 1))
    W2T = jnp.transpose(W2.astype(f32), (0, 2, 1))
    W3T = jnp.pad(jnp.transpose(W3.astype(f32), (0, 2, 1)), ((0, 0), (0, 7), (0, 0)))
    b0c = b0.astype(f32)[:, :, None]
    b1c = b1.astype(f32)[:, :, None]
    b2c = b2.astype(f32)[:, :, None]
    b3c = jnp.pad(b3.astype(f32)[:, :, None], ((0, 0), (0, 7), (0, 0)))

    slab_sizes = (10240, N - 10240)
    feat_parts, stats_parts = [], []
    off = 0
    dr_all = ImageDR.reshape(N, TM, 4).astype(f32)
    nn_all = num_neigh.reshape(N, T).astype(jnp.int32)
    for slab_n in slab_sizes:
        drT = jnp.transpose(dr_all[off:off + slab_n], (2, 1, 0)).reshape(4 * TM, slab_n)
        nnT = nn_all[off:off + slab_n].T
        typs = typT[:, off:off + slab_n]
        grid_a = (slab_n + BN_A - 1) // BN_A
        featT_s, stats_s = pl.pallas_call(
            functools.partial(_stage_a_kernel, slab_n),
            grid=(grid_a,),
            in_specs=[
                pl.BlockSpec((4 * TM, BN_A), lambda i: (0, i)),
                pl.BlockSpec((T, BN_A), lambda i: (0, i)),
                pl.BlockSpec((1, BN_A), lambda i: (0, i)),
                pl.BlockSpec((4 * M1, 4 * T * BETA * 8), lambda i: (0, 0)),
                pl.BlockSpec((4 * M1, 4 * T * BETA * 8), lambda i: (0, 0)),
            ],
            out_specs=[
                pl.BlockSpec((NFEAT, BN_A), lambda i: (0, i)),
                pl.BlockSpec((8, BN_A), lambda i: (0, 0)),
            ],
            out_shape=[
                jax.ShapeDtypeStruct((NFEAT, slab_n), f32),
                jax.ShapeDtypeStruct((8, BN_A), f32),
            ],
        )(drT, nnT, typs, C0, C1)
        feat_parts.append(featT_s)
        stats_parts.append(stats_s)
        off += slab_n
    featT = jnp.concatenate(feat_parts, axis=1)
    stats = jnp.concatenate(stats_parts, axis=0)

    grid_b = (N + BN_B - 1) // BN_B
    ei, etot = pl.pallas_call(
        _stage_b_kernel,
        grid=(grid_b,),
        in_specs=[
            pl.BlockSpec((NFEAT, BN_B), lambda i: (0, i)),
            pl.BlockSpec((1, BN_B), lambda i: (0, i)),
            pl.BlockSpec((16, BN_A), lambda i: (0, 0)),
            pl.BlockSpec((T, 50, NFEAT), lambda i: (0, 0, 0)),
            pl.BlockSpec((T, 50, 1), lambda i: (0, 0, 0)),
            pl.BlockSpec((T, 50, 50), lambda i: (0, 0, 0)),
            pl.BlockSpec((T, 50, 1), lambda i: (0, 0, 0)),
            pl.BlockSpec((T, 50, 50), lambda i: (0, 0, 0)),
            pl.BlockSpec((T, 50, 1), lambda i: (0, 0, 0)),
            pl.BlockSpec((T, 8, 50), lambda i: (0, 0, 0)),
            pl.BlockSpec((T, 8, 1), lambda i: (0, 0, 0)),
        ],
        out_specs=[
            pl.BlockSpec((1, BN_B), lambda i: (0, i)),
            pl.BlockSpec((1, 1), lambda i: (0, 0)),
        ],
        out_shape=[
            jax.ShapeDtypeStruct((1, N), f32),
            jax.ShapeDtypeStruct((1, 1), f32),
        ],
    )(featT, typT, stats, W0T, b0c, W1T, b1c, W2T, b2c, W3T, b3c)

    Ei = ei.reshape(B, N)
    Etot = etot.reshape(B)
    return Etot, Ei


def kernel(list_neigh, Imagetype_map, atom_type, ImageDR, num_neigh, nghost,
           c_param, W0, b0, W1, b1, W2, b2, W3, b3):
    return _run(ImageDR, Imagetype_map, num_neigh, c_param,
                W0, b0, W1, b1, W2, b2, W3, b3)


# speedup vs baseline: 5.1477x; 1.2451x over previous
import functools
import jax
import jax.numpy as jnp
from jax.experimental import pallas as pl
from jax.experimental.pallas import tpu as pltpu

B, N, T, M = 1, 20000, 2, 64
BETA, M1, M2 = 8, 8, 4
RMIN, RMAX = 0.5, 6.0
NFEAT = M1 * M2
TM = T * M

BN_A = 1024
BN_B = 2560


def _stage_a_kernel(slab_n, dr_ref, nn_ref, typ_ref, c0_ref, c1_ref,
                    feat_ref, stats_ref):
    pid = pl.program_id(0)

    @pl.when(pid == 0)
    def _init():
        stats_ref[...] = jnp.zeros_like(stats_ref)

    dr = dr_ref[...]
    bn = dr.shape[1]
    valid = (pid * BN_A + jax.lax.broadcasted_iota(jnp.int32, (1, bn), 1)) < slab_n
    r = dr[0:TM, :]
    dx = dr[TM:2 * TM, :]
    dy = dr[2 * TM:3 * TM, :]
    dz = dr[3 * TM:4 * TM, :]

    row = jax.lax.broadcasted_iota(jnp.int32, (TM, bn), 0)
    nn = jnp.transpose(nn_ref[...], (1, 0))
    nn_sel = jnp.where(row < M, nn[0:1, :], nn[1:2, :])
    mask = ((row % M) < nn_sel).astype(jnp.float32)

    u = (r - RMIN) * (1.0 / (RMAX - RMIN))
    w = jnp.clip(u, 0.0, 1.0) - 0.5
    x = w + w
    pw = jnp.pi * w
    z = pw * pw
    sinpw = pw * (1.0 + z * (-1.0 / 6.0 + z * (1.0 / 120.0 + z * (
        -1.0 / 5040.0 + z * (1.0 / 362880.0 - z / 39916800.0)))))
    fcm = (0.5 - 0.5 * sinpw) * mask

    basis = [fcm, x * fcm]
    tprev, tcur = jnp.ones_like(x), x
    for _ in range(2, BETA):
        tprev, tcur = tcur, 2.0 * x * tcur - tprev
        basis.append(tcur * fcm)

    rinv = 1.0 / jnp.where(r > 1e-6, r, 1.0)
    svec = [None, dx * rinv, dy * rinv, dz * rinv]

    def fold8(p):
        p = p[0:32, :] + p[32:64, :]
        p = p[0:16, :] + p[16:32, :]
        return p[0:8, :] + p[8:16, :]

    g_parts = []
    for c in range(4):
        s = svec[c]
        for t in range(T):
            lo, hi = t * M, (t + 1) * M
            for k in range(BETA):
                p = basis[k][lo:hi, :] if c == 0 else basis[k][lo:hi, :] * s[lo:hi, :]
                g_parts.append(fold8(p))
    G = jnp.concatenate(g_parts, axis=0)

    D0 = jnp.dot(c0_ref[...], G, preferred_element_type=jnp.float32)
    D1 = jnp.dot(c1_ref[...], G, preferred_element_type=jnp.float32)
    typ = jnp.transpose(typ_ref[...], (1, 0))
    D = jnp.where(typ == 0, D0, D1)

    f_parts = []
    for p in range(M2):
        acc = None
        for c in range(4):
            blk = D[c * M1:(c + 1) * M1, :] * D[c * M1 + p:c * M1 + p + 1, :]
            acc = blk if acc is None else acc + blk
        f_parts.append(acc)
    feat = jnp.concatenate(f_parts, axis=0)
    feat = jnp.where(valid, feat, 0.0)
    feat_ref[...] = feat

    colsum = jnp.sum(feat, axis=0, keepdims=True)
    colsq = jnp.sum(feat * feat, axis=0, keepdims=True)
    m0 = jnp.where(valid & (typ == 0), 1.0, 0.0)
    m1 = jnp.where(valid & (typ == 1), 1.0, 0.0)
    upd = jnp.concatenate([
        colsum * m0, colsum * m1,
        colsq * m0, colsq * m1,
        m0, m1,
        jnp.zeros((2, bn), jnp.float32)], axis=0)
    stats_ref[...] += upd


def _stage_b_kernel(feat_ref, typ_ref, stats_ref,
                    w0_ref, b0_ref, w1_ref, b1_ref, w2_ref, b2_ref,
                    w3_ref, b3_ref, ei_ref, etot_ref):
    pid = pl.program_id(0)

    @pl.when(pid == 0)
    def _init():
        etot_ref[...] = jnp.zeros_like(etot_ref)

    stats = stats_ref[...]
    nrow = stats.shape[0]
    def _row(i):
        return sum(jnp.sum(stats[j, :]) for j in range(i, nrow, 8))
    s0 = _row(0)
    s1 = _row(1)
    q0 = _row(2)
    q1 = _row(3)
    c0 = _row(4) * float(NFEAT)
    c1 = _row(5) * float(NFEAT)
    mean0 = s0 / jnp.maximum(c0, 1.0)
    mean1 = s1 / jnp.maximum(c1, 1.0)
    var0 = (q0 - c0 * mean0 * mean0) / jnp.maximum(c0 - 1.0, 1.0)
    var1 = (q1 - c1 * mean1 * mean1) / jnp.maximum(c1 - 1.0, 1.0)
    std0 = jnp.sqrt(jnp.maximum(var0, 0.0))
    std1 = jnp.sqrt(jnp.maximum(var1, 0.0))

    typ = jnp.transpose(typ_ref[...], (1, 0))
    is0 = (typ == 0)
    mean_a = jnp.where(is0, mean0, mean1)
    inv_a = jnp.where(is0, 1.0 / (std0 + 1e-12), 1.0 / (std1 + 1e-12))
    featn = (feat_ref[...] - mean_a) * inv_a

    def mlp(t):
        h = jnp.tanh(jnp.dot(w0_ref[t], featn,
                             preferred_element_type=jnp.float32) + b0_ref[t])
        h = jnp.tanh(jnp.dot(w1_ref[t], h,
                             preferred_element_type=jnp.float32) + b1_ref[t])
        h = jnp.tanh(jnp.dot(w2_ref[t], h,
                             preferred_element_type=jnp.float32) + b2_ref[t])
        return jnp.dot(w3_ref[t], h,
                       preferred_element_type=jnp.float32) + b3_ref[t]

    e0 = mlp(0)
    e1 = mlp(1)
    bn = typ.shape[1]
    valid = (pid * BN_B + jax.lax.broadcasted_iota(jnp.int32, (1, bn), 1)) < N
    ei = jnp.where(is0, e0, jnp.where(typ == 1, e1, 0.0))
    ei = jnp.where(valid, ei, 0.0)
    ei_ref[...] = ei
    etot_ref[...] = etot_ref[...] + jnp.sum(ei, axis=1, keepdims=True)


@jax.jit
def _run(ImageDR, Imagetype_map, num_neigh, c_param,
         W0, b0, W1, b1, W2, b2, W3, b3):
    f32 = jnp.float32
    cp = c_param.astype(f32)
    blk = jnp.transpose(cp, (0, 2, 1, 3)).reshape(T, M1, T * BETA)
    rep = jnp.repeat(blk, 8, axis=2)
    eye4 = jnp.eye(4, dtype=f32)
    Cmat = (eye4[None, :, None, :, None] * rep[:, None, :, None, :])
    Cmat = Cmat.reshape(T, 4 * M1, 4 * T * BETA * 8)
    C0, C1 = Cmat[0], Cmat[1]

    W0p = W0.astype(f32).reshape(T, M1, M2, 50).transpose(0, 2, 1, 3).reshape(T, NFEAT, 50)
    W0T = jnp.transpose(W0p, (0, 2, 1))
    W1T = jnp.transpose(W1.astype(f32), (0, 2, 1))
    W2T = jnp.transpose(W2.astype(f32), (0, 2, 1))
    W3T = jnp.transpose(W3.astype(f32), (0, 2, 1))
    b0c = b0.astype(f32)[:, :, None]
    b1c = b1.astype(f32)[:, :, None]
    b2c = b2.astype(f32)[:, :, None]
    b3c = b3.astype(f32)[:, :, None]

    drT = jnp.transpose(ImageDR.reshape(N, TM, 4).astype(f32), (2, 1, 0))
    drT = drT.reshape(4 * TM, N)
    nn_nat = num_neigh.reshape(N, T).astype(jnp.int32)
    typ_nat = Imagetype_map.astype(jnp.int32).reshape(N, 1)
    grid_a = (N + BN_A - 1) // BN_A
    featT, stats = pl.pallas_call(
        functools.partial(_stage_a_kernel, N),
        grid=(grid_a,),
        in_specs=[
            pl.BlockSpec((4 * TM, BN_A), lambda i: (0, i)),
            pl.BlockSpec((BN_A, T), lambda i: (i, 0)),
            pl.BlockSpec((BN_A, 1), lambda i: (i, 0)),
            pl.BlockSpec((4 * M1, 4 * T * BETA * 8), lambda i: (0, 0)),
            pl.BlockSpec((4 * M1, 4 * T * BETA * 8), lambda i: (0, 0)),
        ],
        out_specs=[
            pl.BlockSpec((NFEAT, BN_A), lambda i: (0, i)),
            pl.BlockSpec((8, BN_A), lambda i: (0, 0)),
        ],
        out_shape=[
            jax.ShapeDtypeStruct((NFEAT, N), f32),
            jax.ShapeDtypeStruct((8, BN_A), f32),
        ],
    )(drT, nn_nat, typ_nat, C0, C1)

    grid_b = (N + BN_B - 1) // BN_B
    ei, etot = pl.pallas_call(
        _stage_b_kernel,
        grid=(grid_b,),
        in_specs=[
            pl.BlockSpec((NFEAT, BN_B), lambda i: (0, i)),
            pl.BlockSpec((BN_B, 1), lambda i: (i, 0)),
            pl.BlockSpec((8, BN_A), lambda i: (0, 0)),
            pl.BlockSpec((T, 50, NFEAT), lambda i: (0, 0, 0)),
            pl.BlockSpec((T, 50, 1), lambda i: (0, 0, 0)),
            pl.BlockSpec((T, 50, 50), lambda i: (0, 0, 0)),
            pl.BlockSpec((T, 50, 1), lambda i: (0, 0, 0)),
            pl.BlockSpec((T, 50, 50), lambda i: (0, 0, 0)),
            pl.BlockSpec((T, 50, 1), lambda i: (0, 0, 0)),
            pl.BlockSpec((T, 1, 50), lambda i: (0, 0, 0)),
            pl.BlockSpec((T, 1, 1), lambda i: (0, 0, 0)),
        ],
        out_specs=[
            pl.BlockSpec((1, BN_B), lambda i: (0, i)),
            pl.BlockSpec((1, 1), lambda i: (0, 0)),
        ],
        out_shape=[
            jax.ShapeDtypeStruct((1, N), f32),
            jax.ShapeDtypeStruct((1, 1), f32),
        ],
    )(featT, typ_nat, stats, W0T, b0c, W1T, b1c, W2T, b2c, W3T, b3c)

    Ei = ei.reshape(B, N)
    Etot = etot.reshape(B)
    return Etot, Ei


def kernel(list_neigh, Imagetype_map, atom_type, ImageDR, num_neigh, nghost,
           c_param, W0, b0, W1, b1, W2, b2, W3, b3):
    return _run(ImageDR, Imagetype_map, num_neigh, c_param,
                W0, b0, W1, b1, W2, b2, W3, b3)


# speedup vs baseline: 5.3947x; 1.0480x over previous
import functools
import jax
import jax.numpy as jnp
from jax.experimental import pallas as pl
from jax.experimental.pallas import tpu as pltpu

B, N, T, M = 1, 20000, 2, 64
BETA, M1, M2 = 8, 8, 4
RMIN, RMAX = 0.5, 6.0
NFEAT = M1 * M2
TM = T * M

BN_A = 2048
BN_B = 5120


def _stage_a_kernel(slab_n, dr_ref, nn_ref, typ_ref, c0_ref, c1_ref,
                    feat_ref, stats_ref):
    pid = pl.program_id(0)

    @pl.when(pid == 0)
    def _init():
        stats_ref[...] = jnp.zeros_like(stats_ref)

    dr = dr_ref[...]
    bn = dr.shape[1]
    valid = (pid * BN_A + jax.lax.broadcasted_iota(jnp.int32, (1, bn), 1)) < slab_n
    r = dr[0:TM, :]
    dx = dr[TM:2 * TM, :]
    dy = dr[2 * TM:3 * TM, :]
    dz = dr[3 * TM:4 * TM, :]

    row = jax.lax.broadcasted_iota(jnp.int32, (TM, bn), 0)
    nn = jnp.transpose(nn_ref[...], (1, 0))
    nn_sel = jnp.where(row < M, nn[0:1, :], nn[1:2, :])
    mask = ((row % M) < nn_sel).astype(jnp.float32)

    u = (r - RMIN) * (1.0 / (RMAX - RMIN))
    w = jnp.clip(u, 0.0, 1.0) - 0.5
    x = w + w
    pw = jnp.pi * w
    z = pw * pw
    sinpw = pw * (1.0 + z * (-1.0 / 6.0 + z * (1.0 / 120.0 + z * (
        -1.0 / 5040.0 + z * (1.0 / 362880.0 - z / 39916800.0)))))
    fcm = (0.5 - 0.5 * sinpw) * mask

    basis = [fcm, x * fcm]
    tprev, tcur = jnp.ones_like(x), x
    for _ in range(2, BETA):
        tprev, tcur = tcur, 2.0 * x * tcur - tprev
        basis.append(tcur * fcm)

    rinv = 1.0 / jnp.where(r > 1e-6, r, 1.0)
    svec = [None, dx * rinv, dy * rinv, dz * rinv]

    def fold8(p):
        p = p[0:32, :] + p[32:64, :]
        p = p[0:16, :] + p[16:32, :]
        return p[0:8, :] + p[8:16, :]

    g_parts = []
    for c in range(4):
        s = svec[c]
        for t in range(T):
            lo, hi = t * M, (t + 1) * M
            for k in range(BETA):
                p = basis[k][lo:hi, :] if c == 0 else basis[k][lo:hi, :] * s[lo:hi, :]
                g_parts.append(fold8(p))
    G = jnp.concatenate(g_parts, axis=0)

    D0 = jnp.dot(c0_ref[...], G, preferred_element_type=jnp.float32)
    D1 = jnp.dot(c1_ref[...], G, preferred_element_type=jnp.float32)
    typ = jnp.transpose(typ_ref[...], (1, 0))
    D = jnp.where(typ == 0, D0, D1)

    f_parts = []
    for p in range(M2):
        acc = None
        for c in range(4):
            blk = D[c * M1:(c + 1) * M1, :] * D[c * M1 + p:c * M1 + p + 1, :]
            acc = blk if acc is None else acc + blk
        f_parts.append(acc)
    feat = jnp.concatenate(f_parts, axis=0)
    feat = jnp.where(valid, feat, 0.0)
    feat_ref[...] = feat

    colsum = jnp.sum(feat, axis=0, keepdims=True)
    colsq = jnp.sum(feat * feat, axis=0, keepdims=True)
    m0 = jnp.where(valid & (typ == 0), 1.0, 0.0)
    m1 = jnp.where(valid & (typ == 1), 1.0, 0.0)
    upd = jnp.concatenate([
        colsum * m0, colsum * m1,
        colsq * m0, colsq * m1,
        m0, m1,
        jnp.zeros((2, bn), jnp.float32)], axis=0)
    stats_ref[...] += upd


def _stage_b_kernel(feat_ref, typ_ref, stats_ref,
                    w0_ref, b0_ref, w1_ref, b1_ref, w2_ref, b2_ref,
                    w3_ref, b3_ref, ei_ref, etot_ref):
    pid = pl.program_id(0)

    @pl.when(pid == 0)
    def _init():
        etot_ref[...] = jnp.zeros_like(etot_ref)

    stats = stats_ref[...]
    nrow = stats.shape[0]
    def _row(i):
        return sum(jnp.sum(stats[j, :]) for j in range(i, nrow, 8))
    s0 = _row(0)
    s1 = _row(1)
    q0 = _row(2)
    q1 = _row(3)
    c0 = _row(4) * float(NFEAT)
    c1 = _row(5) * float(NFEAT)
    mean0 = s0 / jnp.maximum(c0, 1.0)
    mean1 = s1 / jnp.maximum(c1, 1.0)
    var0 = (q0 - c0 * mean0 * mean0) / jnp.maximum(c0 - 1.0, 1.0)
    var1 = (q1 - c1 * mean1 * mean1) / jnp.maximum(c1 - 1.0, 1.0)
    std0 = jnp.sqrt(jnp.maximum(var0, 0.0))
    std1 = jnp.sqrt(jnp.maximum(var1, 0.0))

    typ = jnp.transpose(typ_ref[...], (1, 0))
    is0 = (typ == 0)
    mean_a = jnp.where(is0, mean0, mean1)
    inv_a = jnp.where(is0, 1.0 / (std0 + 1e-12), 1.0 / (std1 + 1e-12))
    featn = (feat_ref[...] - mean_a) * inv_a

    def mlp(t):
        h = jnp.tanh(jnp.dot(w0_ref[t], featn,
                             preferred_element_type=jnp.float32) + b0_ref[t])
        h = jnp.tanh(jnp.dot(jnp.transpose(w1_ref[t], (1, 0)), h,
                             preferred_element_type=jnp.float32) + b1_ref[t])
        h = jnp.tanh(jnp.dot(jnp.transpose(w2_ref[t], (1, 0)), h,
                             preferred_element_type=jnp.float32) + b2_ref[t])
        return jnp.dot(w3_ref[t], h,
                       preferred_element_type=jnp.float32) + b3_ref[t]

    e0 = mlp(0)
    e1 = mlp(1)
    bn = typ.shape[1]
    valid = (pid * BN_B + jax.lax.broadcasted_iota(jnp.int32, (1, bn), 1)) < N
    ei = jnp.where(is0, e0, jnp.where(typ == 1, e1, 0.0))
    ei = jnp.where(valid, ei, 0.0)
    ei_ref[...] = ei
    etot_ref[...] = etot_ref[...] + jnp.sum(ei, axis=1, keepdims=True)


@jax.jit
def _run(ImageDR, Imagetype_map, num_neigh, c_param,
         W0, b0, W1, b1, W2, b2, W3, b3):
    f32 = jnp.float32
    cp = c_param.astype(f32)
    blk = jnp.transpose(cp, (0, 2, 1, 3)).reshape(T, M1, T * BETA)
    rep = jnp.repeat(blk, 8, axis=2)
    eye4 = jnp.eye(4, dtype=f32)
    Cmat = (eye4[None, :, None, :, None] * rep[:, None, :, None, :])
    Cmat = Cmat.reshape(T, 4 * M1, 4 * T * BETA * 8)
    C0, C1 = Cmat[0], Cmat[1]

    W0p = W0.astype(f32).reshape(T, M1, M2, 50).transpose(0, 2, 1, 3).reshape(T, NFEAT, 50)
    W0T = jnp.transpose(W0p, (0, 2, 1))
    W1T = W1.astype(f32)
    W2T = W2.astype(f32)
    W3T = jnp.transpose(W3.astype(f32), (0, 2, 1))
    b0c = b0.astype(f32)[:, :, None]
    b1c = b1.astype(f32)[:, :, None]
    b2c = b2.astype(f32)[:, :, None]
    b3c = b3.astype(f32)[:, :, None]

    drT = jnp.transpose(ImageDR.reshape(N, TM, 4).astype(f32), (2, 1, 0))
    drT = drT.reshape(4 * TM, N)
    nn_nat = num_neigh.reshape(N, T).astype(jnp.int32)
    typ_nat = Imagetype_map.astype(jnp.int32).reshape(N, 1)
    grid_a = (N + BN_A - 1) // BN_A
    featT, stats = pl.pallas_call(
        functools.partial(_stage_a_kernel, N),
        grid=(grid_a,),
        in_specs=[
            pl.BlockSpec((4 * TM, BN_A), lambda i: (0, i)),
            pl.BlockSpec((BN_A, T), lambda i: (i, 0)),
            pl.BlockSpec((BN_A, 1), lambda i: (i, 0)),
            pl.BlockSpec((4 * M1, 4 * T * BETA * 8), lambda i: (0, 0)),
            pl.BlockSpec((4 * M1, 4 * T * BETA * 8), lambda i: (0, 0)),
        ],
        out_specs=[
            pl.BlockSpec((NFEAT, BN_A), lambda i: (0, i)),
            pl.BlockSpec((8, BN_A), lambda i: (0, 0)),
        ],
        out_shape=[
            jax.ShapeDtypeStruct((NFEAT, N), f32),
            jax.ShapeDtypeStruct((8, BN_A), f32),
        ],
    )(drT, nn_nat, typ_nat, C0, C1)

    grid_b = (N + BN_B - 1) // BN_B
    ei, etot = pl.pallas_call(
        _stage_b_kernel,
        grid=(grid_b,),
        in_specs=[
            pl.BlockSpec((NFEAT, BN_B), lambda i: (0, i)),
            pl.BlockSpec((BN_B, 1), lambda i: (i, 0)),
            pl.BlockSpec((8, BN_A), lambda i: (0, 0)),
            pl.BlockSpec((T, 50, NFEAT), lambda i: (0, 0, 0)),
            pl.BlockSpec((T, 50, 1), lambda i: (0, 0, 0)),
            pl.BlockSpec((T, 50, 50), lambda i: (0, 0, 0)),
            pl.BlockSpec((T, 50, 1), lambda i: (0, 0, 0)),
            pl.BlockSpec((T, 50, 50), lambda i: (0, 0, 0)),
            pl.BlockSpec((T, 50, 1), lambda i: (0, 0, 0)),
            pl.BlockSpec((T, 1, 50), lambda i: (0, 0, 0)),
            pl.BlockSpec((T, 1, 1), lambda i: (0, 0, 0)),
        ],
        out_specs=[
            pl.BlockSpec((1, BN_B), lambda i: (0, i)),
            pl.BlockSpec((1, 1), lambda i: (0, 0)),
        ],
        out_shape=[
            jax.ShapeDtypeStruct((1, N), f32),
            jax.ShapeDtypeStruct((1, 1), f32),
        ],
    )(featT, typ_nat, stats, W0T, b0c, W1T, b1c, W2T, b2c, W3T, b3c)

    Ei = ei.reshape(B, N)
    Etot = etot.reshape(B)
    return Etot, Ei


def kernel(list_neigh, Imagetype_map, atom_type, ImageDR, num_neigh, nghost,
           c_param, W0, b0, W1, b1, W2, b2, W3, b3):
    return _run(ImageDR, Imagetype_map, num_neigh, c_param,
                W0, b0, W1, b1, W2, b2, W3, b3)
